# Initial kernel scaffold; baseline (speedup 1.0000x reference)
import jax
import jax.numpy as jnp
from jax.experimental import pallas as pl


def kernel(inputs):
    raise NotImplementedError("write your pallas kernel here")


# speedup vs baseline: 2.0733x; 2.0733x over previous
import functools

import jax
import jax.numpy as jnp
from jax.experimental import pallas as pl

NBINS = 256
H = 512
W = 512
C = 3
ROWCHUNK = 8


def _body(x_ref, wg_ref, wr_ref, o_ref):
    xi = x_ref[0]
    gray = jnp.dot(xi.astype(jnp.bfloat16), wg_ref[...],
                   preferred_element_type=jnp.float32)
    gmin = jnp.min(gray)
    gmax = jnp.max(gray)
    scale = jnp.max(
        NBINS / jnp.maximum(jnp.full((8, 128), gmax - gmin, jnp.float32), 1e-12))
    idx = jnp.clip(((gray - gmin) * scale).astype(jnp.int32), 0, NBINS - 1)

    bins3 = jax.lax.broadcasted_iota(jnp.int32, (NBINS, 1, 1), 0)
    hist2d = jnp.zeros((NBINS, W), jnp.float32)
    for c in range(H // ROWCHUNK):
        chunk = idx[c * ROWCHUNK:(c + 1) * ROWCHUNK, :]
        oh = (chunk[None, :, :] == bins3).astype(jnp.float32)
        hist2d = hist2d + jnp.sum(oh, axis=1)
    hist = jax.lax.dot_general(
        jnp.ones((1, W), jnp.float32), hist2d,
        (((1,), (1,)), ((), ())),
        preferred_element_type=jnp.float32,
        precision=jax.lax.Precision.HIGHEST)

    iint = jax.lax.broadcasted_iota(jnp.int32, (1, NBINS), 1)
    ii = iint.astype(jnp.float32)
    centers = gmin + (ii + 0.5) / scale
    wc = hist * centers

    li = jax.lax.broadcasted_iota(jnp.int32, (NBINS, NBINS), 0)
    lj = jax.lax.broadcasted_iota(jnp.int32, (NBINS, NBINS), 1)
    lt = (li <= lj).astype(jnp.float32)
    w1 = jnp.dot(hist, lt, preferred_element_type=jnp.float32,
                 precision=jax.lax.Precision.HIGHEST)

    def c1_step(j, carry):
        acc, acc2, c1v = carry
        wcj = jnp.sum(jnp.where(iint == j, wc, 0.0))
        in_lo = j < (NBINS // 2)
        acc = jnp.where(in_lo, acc + wcj, acc)
        acc2 = jnp.where(in_lo, acc2, acc2 + wcj)
        c1j = jnp.where(in_lo, acc, acc2 + acc)
        c1v = c1v + jnp.where(iint == j, c1j, 0.0)
        return acc, acc2, c1v

    _, _, c1 = jax.lax.fori_loop(
        0, NBINS, c1_step,
        (jnp.float32(0), jnp.float32(0), jnp.zeros((1, NBINS), jnp.float32)))

    total_w = jnp.sum(jnp.where(iint == NBINS - 1, w1, 0.0))
    total_c = jnp.sum(jnp.where(iint == NBINS - 1, c1, 0.0))
    w2p = total_w - w1
    m1 = c1 / jnp.maximum(w1, 1e-12)
    m2 = (total_c - c1) / jnp.maximum(w2p, 1e-12)
    var12 = w1 * w2p * (m1 - m2) ** 2
    var12 = jnp.where(iint < NBINS - 1, var12, -jnp.inf)
    vmax = jnp.max(var12)
    pick = jnp.min(jnp.where(var12 == vmax, ii, jnp.float32(NBINS)))
    thr = jnp.sum(jnp.where(ii == pick, centers, 0.0))

    binary = jnp.where(gray > thr, 255.0, 0.0)
    o_ref[0] = jnp.dot(binary, wr_ref[...], preferred_element_type=jnp.float32)


@functools.partial(jax.jit, static_argnames=("interpret",))
def kernel(inputs, interpret=False):
    B = inputs.shape[0]
    x = inputs.reshape(B, H, W * C)
    wvec = jnp.array([0.2989, 0.587, 0.114], jnp.float32)
    rows = jnp.arange(W * C)
    cols = jnp.arange(W)
    wg = jnp.where((rows[:, None] // C) == cols[None, :],
                   wvec[rows % C][:, None], 0.0).astype(jnp.bfloat16)
    wr = ((rows[None, :] // C) == cols[:, None]).astype(jnp.float32)
    out = pl.pallas_call(
        _body,
        grid=(B,),
        in_specs=[
            pl.BlockSpec((1, H, W * C), lambda b: (b, 0, 0)),
            pl.BlockSpec((W * C, W), lambda b: (0, 0)),
            pl.BlockSpec((W, W * C), lambda b: (0, 0)),
        ],
        out_specs=pl.BlockSpec((1, H, W * C), lambda b: (b, 0, 0)),
        out_shape=jax.ShapeDtypeStruct((B, H, W * C), jnp.float32),
        interpret=interpret,
    )(x, wg, wr)
    return out.reshape(B, H, W, C)


# speedup vs baseline: 15.8773x; 7.6581x over previous
import functools

import jax
import jax.numpy as jnp
from jax import lax
from jax.experimental import pallas as pl
from jax.experimental.pallas import tpu as pltpu
from jax.experimental.pallas import tpu_sc as plsc

NBINS = 256
H = 512
W = 512
C = 3
B = 32
NW = 32
NLANE = 16
PIX = H * W
SC_CHUNK = 65536


def _stage_a(x_ref, wg_ref, gray_ref, idx_ref, stats_ref):
    xi = x_ref[0]
    gray = jnp.dot(xi.astype(jnp.bfloat16), wg_ref[...],
                   preferred_element_type=jnp.float32)
    gray_ref[0] = gray
    gmin = jnp.min(gray)
    gmax = jnp.max(gray)
    scale = jnp.max(
        NBINS / jnp.maximum(jnp.full((8, 128), gmax - gmin, jnp.float32), 1e-12))
    idx = jnp.clip(((gray - gmin) * scale).astype(jnp.int32), 0, NBINS - 1)
    lane = jax.lax.broadcasted_iota(jnp.int32, (H, W), 1) & (NLANE - 1)
    idx_ref[0] = idx + (lane << 8)
    li = jax.lax.broadcasted_iota(jnp.int32, (8, 128), 1)
    stats_ref[0] = jnp.where(li == 0, gmin, jnp.where(li == 1, gmax, 0.0))


def _sc_hist(idx_hbm, out_hbm, buf, hist):
    cid = lax.axis_index("c")
    sid = lax.axis_index("s")
    w = sid * 2 + cid
    zero = jnp.zeros((NLANE,), jnp.float32)
    ones = jnp.full((NLANE,), 1.0, jnp.float32)

    def zstep(i, _):
        hist[pl.ds(i * NLANE, NLANE)] = zero
        return 0

    lax.fori_loop(0, (NLANE * NBINS) // NLANE, zstep, 0)

    def scat(i, _):
        v = buf[pl.ds(i * NLANE, NLANE)]
        plsc.addupdate_scatter(hist, [v], ones)
        return 0

    for ci in range(PIX // SC_CHUNK):
        pltpu.sync_copy(idx_hbm.at[w, pl.ds(ci * SC_CHUNK, SC_CHUNK)], buf)
        lax.fori_loop(0, SC_CHUNK // NLANE, scat, 0)
    pltpu.sync_copy(hist, out_hbm.at[w])


def _stage_c1(h_ref, stats_ref, thr_ref):
    h = h_ref[...]
    hist = jnp.zeros((B, NBINS), jnp.float32)
    for l in range(NLANE):
        hist = hist + h[:, l * NBINS:(l + 1) * NBINS]

    stats = stats_ref[...][:, 0, :]
    gmin = stats[:, 0:1]
    gmax = stats[:, 1:2]
    scale = NBINS / jnp.maximum(gmax - gmin, 1e-12)

    iint = jax.lax.broadcasted_iota(jnp.int32, (1, NBINS), 1)
    ii = iint.astype(jnp.float32)
    centers = gmin + (ii + 0.5) / scale
    wc = hist * centers

    li = jax.lax.broadcasted_iota(jnp.int32, (NBINS, NBINS), 0)
    lj = jax.lax.broadcasted_iota(jnp.int32, (NBINS, NBINS), 1)
    lt = (li <= lj).astype(jnp.float32)
    w1 = jnp.dot(hist, lt, preferred_element_type=jnp.float32,
                 precision=jax.lax.Precision.HIGHEST)

    def c1_step(j, carry):
        acc, acc2, c1v = carry
        wcj = jnp.sum(jnp.where(iint == j, wc, 0.0), axis=1, keepdims=True)
        in_lo = j < (NBINS // 2)
        acc = jnp.where(in_lo, acc + wcj, acc)
        acc2 = jnp.where(in_lo, acc2, acc2 + wcj)
        c1j = jnp.where(in_lo, acc, acc2 + acc)
        c1v = c1v + jnp.where(iint == j, c1j, 0.0)
        return acc, acc2, c1v

    zcol = jnp.zeros((B, 1), jnp.float32)
    _, _, c1 = lax.fori_loop(
        0, NBINS, c1_step, (zcol, zcol, jnp.zeros((B, NBINS), jnp.float32)))

    total_w = jnp.sum(jnp.where(iint == NBINS - 1, w1, 0.0), axis=1, keepdims=True)
    total_c = jnp.sum(jnp.where(iint == NBINS - 1, c1, 0.0), axis=1, keepdims=True)
    w2p = total_w - w1
    m1 = c1 / jnp.maximum(w1, 1e-12)
    m2 = (total_c - c1) / jnp.maximum(w2p, 1e-12)
    var12 = w1 * w2p * (m1 - m2) ** 2
    var12 = jnp.where(iint < NBINS - 1, var12, -jnp.inf)
    vmax = jnp.max(var12, axis=1, keepdims=True)
    pick = jnp.min(jnp.where(var12 == vmax, ii, jnp.float32(NBINS)),
                   axis=1, keepdims=True)
    thr = jnp.sum(jnp.where(ii == pick, centers, 0.0), axis=1, keepdims=True)
    thr_ref[...] = jnp.broadcast_to(thr[:, :, None], (B, 8, 128))


def _stage_c2(gray_ref, thr_ref, wr_ref, o_ref):
    thr = jnp.max(thr_ref[0])
    binary = jnp.where(gray_ref[0] > thr, 255.0, 0.0)
    o_ref[0] = jnp.dot(binary, wr_ref[...], preferred_element_type=jnp.float32)


@jax.jit
def kernel(inputs):
    x = inputs.reshape(B, H, W * C)
    wvec = jnp.array([0.2989, 0.587, 0.114], jnp.float32)
    rows = jnp.arange(W * C)
    cols = jnp.arange(W)
    wg = jnp.where((rows[:, None] // C) == cols[None, :],
                   wvec[rows % C][:, None], 0.0).astype(jnp.bfloat16)
    wr = ((rows[None, :] // C) == cols[:, None]).astype(jnp.float32)

    gray, idxsc, stats = pl.pallas_call(
        _stage_a,
        grid=(B,),
        in_specs=[
            pl.BlockSpec((1, H, W * C), lambda b: (b, 0, 0)),
            pl.BlockSpec((W * C, W), lambda b: (0, 0)),
        ],
        out_specs=[
            pl.BlockSpec((1, H, W), lambda b: (b, 0, 0)),
            pl.BlockSpec((1, H, W), lambda b: (b, 0, 0)),
            pl.BlockSpec((1, 8, 128), lambda b: (b, 0, 0)),
        ],
        out_shape=[
            jax.ShapeDtypeStruct((B, H, W), jnp.float32),
            jax.ShapeDtypeStruct((B, H, W), jnp.int32),
            jax.ShapeDtypeStruct((B, 8, 128), jnp.float32),
        ],
    )(x, wg)

    hist16 = pl.kernel(
        _sc_hist,
        out_type=jax.ShapeDtypeStruct((B, NLANE * NBINS), jnp.float32),
        mesh=plsc.VectorSubcoreMesh(core_axis_name="c", subcore_axis_name="s"),
        compiler_params=pltpu.CompilerParams(needs_layout_passes=False),
        scratch_types=[
            pltpu.VMEM((SC_CHUNK,), jnp.int32),
            pltpu.VMEM((NLANE * NBINS,), jnp.float32),
        ],
    )(idxsc.reshape(B, PIX))

    thr = pl.pallas_call(
        _stage_c1,
        grid=(1,),
        in_specs=[
            pl.BlockSpec((B, NLANE * NBINS), lambda b: (0, 0)),
            pl.BlockSpec((B, 8, 128), lambda b: (0, 0, 0)),
        ],
        out_specs=pl.BlockSpec((B, 8, 128), lambda b: (0, 0, 0)),
        out_shape=jax.ShapeDtypeStruct((B, 8, 128), jnp.float32),
    )(hist16, stats)

    out = pl.pallas_call(
        _stage_c2,
        grid=(B,),
        in_specs=[
            pl.BlockSpec((1, H, W), lambda b: (b, 0, 0)),
            pl.BlockSpec((1, 8, 128), lambda b: (b, 0, 0)),
            pl.BlockSpec((W, W * C), lambda b: (0, 0)),
        ],
        out_specs=pl.BlockSpec((1, H, W * C), lambda b: (b, 0, 0)),
        out_shape=jax.ShapeDtypeStruct((B, H, W * C), jnp.float32),
    )(gray, thr, wr)
    return out.reshape(B, H, W, C)


# speedup vs baseline: 17.0708x; 1.0752x over previous
import functools

import jax
import jax.numpy as jnp
from jax import lax
from jax.experimental import pallas as pl
from jax.experimental.pallas import tpu as pltpu
from jax.experimental.pallas import tpu_sc as plsc

NBINS = 256
H = 512
W = 512
C = 3
B = 32
NW = 32
NLANE = 16
PIX = H * W
SC_ROWS = 64


def _stage_a(x_ref, wg_ref, gray_ref, idx_ref, stats_ref):
    xi = x_ref[0]
    gray = jnp.dot(xi.astype(jnp.bfloat16), wg_ref[...],
                   preferred_element_type=jnp.float32)
    gray_ref[0] = gray
    gmin = jnp.min(gray)
    gmax = jnp.max(gray)
    scale = jnp.max(
        NBINS / jnp.maximum(jnp.full((8, 128), gmax - gmin, jnp.float32), 1e-12))
    idx = jnp.clip(((gray - gmin) * scale).astype(jnp.int32), 0, NBINS - 1)
    lane = jax.lax.broadcasted_iota(jnp.int32, (H, W), 1) & (NLANE - 1)
    idx_ref[0] = idx + (lane << 8)
    li = jax.lax.broadcasted_iota(jnp.int32, (8, 128), 1)
    stats_ref[0] = jnp.where(li == 0, gmin, jnp.where(li == 1, gmax, 0.0))


def _sc_hist(idx_hbm, out_hbm, buf0, buf1, hist, sem0, sem1):
    cid = lax.axis_index("c")
    sid = lax.axis_index("s")
    w = sid * 2 + cid
    zero = jnp.zeros((NLANE,), jnp.float32)
    ones = jnp.full((NLANE,), 1.0, jnp.float32)

    def zstep(i, _):
        hist[pl.ds(i * NLANE, NLANE)] = zero
        return 0

    lax.fori_loop(0, (NLANE * NBINS) // NLANE, zstep, 0)

    bufs = (buf0, buf1)
    sems = (sem0, sem1)
    nchunk = H // SC_ROWS
    copies = []
    for ci in range(nchunk):
        copies.append(pltpu.make_async_copy(
            idx_hbm.at[w, pl.ds(ci * SC_ROWS, SC_ROWS), :],
            bufs[ci % 2], sems[ci % 2]))
    copies[0].start()
    for ci in range(nchunk):
        if ci + 1 < nchunk:
            copies[ci + 1].start()
        copies[ci].wait()
        buf = bufs[ci % 2]

        def scat(i, _):
            r = i >> 5
            c = i & 31
            v = buf[r, pl.ds(c * NLANE, NLANE)]
            plsc.addupdate_scatter(hist, [v], ones)
            return 0

        lax.fori_loop(0, (SC_ROWS * W) // NLANE, scat, 0)
    pltpu.sync_copy(hist, out_hbm.at[w])


def _stage_c1(h_ref, stats_ref, thr_ref):
    h = h_ref[...]
    hist = jnp.zeros((B, NBINS), jnp.float32)
    for l in range(NLANE):
        hist = hist + h[:, l * NBINS:(l + 1) * NBINS]

    stats = stats_ref[...][:, 0, :]
    gmin = stats[:, 0:1]
    gmax = stats[:, 1:2]
    scale = NBINS / jnp.maximum(gmax - gmin, 1e-12)

    iint = jax.lax.broadcasted_iota(jnp.int32, (1, NBINS), 1)
    ii = iint.astype(jnp.float32)
    centers = gmin + (ii + 0.5) / scale
    wc = hist * centers

    li = jax.lax.broadcasted_iota(jnp.int32, (NBINS, NBINS), 0)
    lj = jax.lax.broadcasted_iota(jnp.int32, (NBINS, NBINS), 1)
    lt = (li <= lj).astype(jnp.float32)
    w1 = jnp.dot(hist, lt, preferred_element_type=jnp.float32,
                 precision=jax.lax.Precision.HIGHEST)

    def c1_step(j, carry):
        acc, acc2, c1v = carry
        wcj = jnp.sum(jnp.where(iint == j, wc, 0.0), axis=1, keepdims=True)
        in_lo = j < (NBINS // 2)
        acc = jnp.where(in_lo, acc + wcj, acc)
        acc2 = jnp.where(in_lo, acc2, acc2 + wcj)
        c1j = jnp.where(in_lo, acc, acc2 + acc)
        c1v = c1v + jnp.where(iint == j, c1j, 0.0)
        return acc, acc2, c1v

    zcol = jnp.zeros((B, 1), jnp.float32)
    _, _, c1 = lax.fori_loop(
        0, NBINS, c1_step, (zcol, zcol, jnp.zeros((B, NBINS), jnp.float32)))

    total_w = jnp.sum(jnp.where(iint == NBINS - 1, w1, 0.0), axis=1, keepdims=True)
    total_c = jnp.sum(jnp.where(iint == NBINS - 1, c1, 0.0), axis=1, keepdims=True)
    w2p = total_w - w1
    m1 = c1 / jnp.maximum(w1, 1e-12)
    m2 = (total_c - c1) / jnp.maximum(w2p, 1e-12)
    var12 = w1 * w2p * (m1 - m2) ** 2
    var12 = jnp.where(iint < NBINS - 1, var12, -jnp.inf)
    vmax = jnp.max(var12, axis=1, keepdims=True)
    pick = jnp.min(jnp.where(var12 == vmax, ii, jnp.float32(NBINS)),
                   axis=1, keepdims=True)
    thr = jnp.sum(jnp.where(ii == pick, centers, 0.0), axis=1, keepdims=True)
    thr_ref[...] = jnp.broadcast_to(thr[:, :, None], (B, 8, 128))


def _stage_c2(gray_ref, thr_ref, wr_ref, o_ref):
    thr = jnp.max(thr_ref[0])
    binary = jnp.where(gray_ref[0] > thr, 255.0, 0.0)
    o_ref[0] = jnp.dot(binary, wr_ref[...], preferred_element_type=jnp.float32)


@jax.jit
def kernel(inputs):
    x = inputs.reshape(B, H, W * C)
    wvec = jnp.array([0.2989, 0.587, 0.114], jnp.float32)
    rows = jnp.arange(W * C)
    cols = jnp.arange(W)
    wg = jnp.where((rows[:, None] // C) == cols[None, :],
                   wvec[rows % C][:, None], 0.0).astype(jnp.bfloat16)
    wr = ((rows[None, :] // C) == cols[:, None]).astype(jnp.float32)

    gray, idxsc, stats = pl.pallas_call(
        _stage_a,
        grid=(B,),
        in_specs=[
            pl.BlockSpec((1, H, W * C), lambda b: (b, 0, 0)),
            pl.BlockSpec((W * C, W), lambda b: (0, 0)),
        ],
        out_specs=[
            pl.BlockSpec((1, H, W), lambda b: (b, 0, 0)),
            pl.BlockSpec((1, H, W), lambda b: (b, 0, 0)),
            pl.BlockSpec((1, 8, 128), lambda b: (b, 0, 0)),
        ],
        out_shape=[
            jax.ShapeDtypeStruct((B, H, W), jnp.float32),
            jax.ShapeDtypeStruct((B, H, W), jnp.int32),
            jax.ShapeDtypeStruct((B, 8, 128), jnp.float32),
        ],
    )(x, wg)

    hist16 = pl.kernel(
        _sc_hist,
        out_type=jax.ShapeDtypeStruct((B, NLANE * NBINS), jnp.float32),
        mesh=plsc.VectorSubcoreMesh(core_axis_name="c", subcore_axis_name="s"),
        compiler_params=pltpu.CompilerParams(needs_layout_passes=False),
        scratch_types=[
            pltpu.VMEM((SC_ROWS, W), jnp.int32),
            pltpu.VMEM((SC_ROWS, W), jnp.int32),
            pltpu.VMEM((NLANE * NBINS,), jnp.float32),
            pltpu.SemaphoreType.DMA,
            pltpu.SemaphoreType.DMA,
        ],
    )(idxsc)

    thr = pl.pallas_call(
        _stage_c1,
        grid=(1,),
        in_specs=[
            pl.BlockSpec((B, NLANE * NBINS), lambda b: (0, 0)),
            pl.BlockSpec((B, 8, 128), lambda b: (0, 0, 0)),
        ],
        out_specs=pl.BlockSpec((B, 8, 128), lambda b: (0, 0, 0)),
        out_shape=jax.ShapeDtypeStruct((B, 8, 128), jnp.float32),
    )(hist16, stats)

    out = pl.pallas_call(
        _stage_c2,
        grid=(B,),
        in_specs=[
            pl.BlockSpec((1, H, W), lambda b: (b, 0, 0)),
            pl.BlockSpec((1, 8, 128), lambda b: (b, 0, 0)),
            pl.BlockSpec((W, W * C), lambda b: (0, 0)),
        ],
        out_specs=pl.BlockSpec((1, H, W * C), lambda b: (b, 0, 0)),
        out_shape=jax.ShapeDtypeStruct((B, H, W * C), jnp.float32),
    )(gray, thr, wr)
    return out.reshape(B, H, W, C)
